# baseline (speedup 1.0000x reference)
import functools

import jax
import jax.numpy as jnp
from jax import lax
from jax.experimental import pallas as pl
from jax.experimental.pallas import tpu as pltpu
from jax.experimental.pallas import tpu_sc as plsc

_N = 100000
_D = 500
_MD = 100
_B = 16384

_NC = 2
_NS = 16
_NW = _NC * _NS
_BPW = _B // _NW
_CRM = 32
_NRM = _BPW // _CRM
_SCAN = 1024
_NSB = _B // _SCAN
_CHB = 64
_NCHB = _BPW // _CHB

_SCMESH = dict(core_axis_name="c", subcore_axis_name="s",
               num_cores=_NC, num_subcores=_NS)


def _sc_winner_body(rm_hbm, idx_hbm, rm2_out, idx_v, w_v, tab, scan_v, rmbuf,
                    sem_s, sem_r, sem_rw):
    wid = lax.axis_index("s") * _NC + lax.axis_index("c")
    base = wid * _BPW
    pltpu.sync_copy(idx_hbm.at[pl.ds(base, _BPW)], idx_v)
    lanes = jnp.arange(16, dtype=jnp.int32)

    def _scan_compute(b, p):
        def _chunk(q, _):
            v = scan_v[p, pl.ds(q * 16, 16)]
            j = b * _SCAN + q * 16 + lanes
            plsc.store_scatter(tab, [v], j)

            def _wbody(c):
                got = plsc.load_gather(tab, [v])
                msk = got < j
                plsc.store_scatter(tab, [v], j, mask=msk)
                return plsc.all_reduce_population_count(msk)[0]

            lax.while_loop(lambda c: c > 0, _wbody, jnp.int32(1))
            return 0

        lax.fori_loop(0, _SCAN // 16, _chunk, 0)

    s_cps = {0: pltpu.async_copy(idx_hbm.at[pl.ds(0, _SCAN)],
                                 scan_v.at[0], sem_s)}
    for b in range(_NSB):
        s_cps.pop(b).wait()
        if b + 1 < _NSB:
            s_cps[b + 1] = pltpu.async_copy(
                idx_hbm.at[pl.ds((b + 1) * _SCAN, _SCAN)],
                scan_v.at[(b + 1) % 2], sem_s)
        _scan_compute(b, b % 2)

    def _wq(q, _):
        w_v[pl.ds(q * 16, 16)] = plsc.load_gather(
            tab, [idx_v[pl.ds(q * 16, 16)]])
        return 0

    lax.fori_loop(0, _BPW // 16, _wq, 0)

    def _issue_rm(c):
        return pltpu.async_copy(
            rm_hbm.at[w_v.at[pl.ds(c * _CRM, _CRM)]], rmbuf.at[c % 2], sem_r)

    r_cps = {0: _issue_rm(0)}
    rw_cps = {}
    for c in range(_NRM):
        if c + 1 < _NRM:
            r_cps[c + 1] = _issue_rm(c + 1)
        r_cps.pop(c).wait()
        rw_cps[c] = pltpu.async_copy(
            rmbuf.at[c % 2], rm2_out.at[pl.ds(base + c * _CRM, _CRM)], sem_rw)
        if c - 1 in rw_cps:
            rw_cps.pop(c - 1).wait()
    rw_cps.pop(_NRM - 1).wait()


@functools.cache
def _sc_winner():
    return pl.kernel(
        _sc_winner_body,
        out_type=jax.ShapeDtypeStruct((_B, 128), jnp.float32),
        mesh=plsc.VectorSubcoreMesh(**_SCMESH),
        compiler_params=pltpu.CompilerParams(needs_layout_passes=False),
        scratch_types=[
            pltpu.VMEM((_BPW,), jnp.int32),
            pltpu.VMEM((_BPW,), jnp.int32),
            pltpu.VMEM((_N,), jnp.int32),
            pltpu.VMEM((2, _SCAN), jnp.int32),
            pltpu.VMEM((2, _CRM, 128), jnp.float32),
            pltpu.SemaphoreType.DMA,
            pltpu.SemaphoreType.DMA,
            pltpu.SemaphoreType.DMA,
        ],
    )


def _sc_hgather_body(mem_hbm, idx_hbm, h_out, idx_v, hbuf, sem_g, sem_w):
    wid = lax.axis_index("s") * _NC + lax.axis_index("c")
    base = wid * _BPW
    pltpu.sync_copy(idx_hbm.at[pl.ds(base, _BPW)], idx_v)

    def _issue(c):
        return pltpu.async_copy(
            mem_hbm.at[idx_v.at[pl.ds(c * _CHB, _CHB)]], hbuf.at[c % 2], sem_g)

    g_cps = {0: _issue(0)}
    w_cps = {}
    for c in range(_NCHB):
        if c + 1 < _NCHB:
            g_cps[c + 1] = _issue(c + 1)
        g_cps.pop(c).wait()
        w_cps[c] = pltpu.async_copy(
            hbuf.at[c % 2], h_out.at[pl.ds(base + c * _CHB, _CHB)], sem_w)
        if c - 1 in w_cps:
            w_cps.pop(c - 1).wait()
    w_cps.pop(_NCHB - 1).wait()


@functools.cache
def _sc_hgather():
    return pl.kernel(
        _sc_hgather_body,
        out_type=jax.ShapeDtypeStruct((_B, 512), jnp.float32),
        mesh=plsc.VectorSubcoreMesh(**_SCMESH),
        compiler_params=pltpu.CompilerParams(needs_layout_passes=False),
        scratch_types=[
            pltpu.VMEM((_BPW,), jnp.int32),
            pltpu.VMEM((2, _CHB, 512), jnp.float32),
            pltpu.SemaphoreType.DMA,
            pltpu.SemaphoreType.DMA,
        ],
    )


_BM = 256


def _tc_body(h_ref, rm_ref, w1_ref, b1_ref, w2_ref, b2_ref,
             wir_ref, wiz_ref, win_ref, bi_ref,
             whr_ref, whz_ref, whn_ref, bh_ref, out_ref):
    f32 = jnp.float32
    bf16 = jnp.bfloat16
    cdims = (((1,), (1,)), ((), ()))
    h = h_ref[:, :_D]
    hb = h.astype(bf16)
    m = jax.nn.relu(
        lax.dot_general(rm_ref[:, :_MD], w1_ref[...], cdims,
                        preferred_element_type=f32) + b1_ref[...])
    m = lax.dot_general(m, w2_ref[...], cdims,
                        preferred_element_type=f32) + b2_ref[...]
    mb = m.astype(bf16)
    gir = lax.dot_general(mb, wir_ref[...], cdims,
                          preferred_element_type=f32) + bi_ref[0:1, :]
    giz = lax.dot_general(mb, wiz_ref[...], cdims,
                          preferred_element_type=f32) + bi_ref[1:2, :]
    gin = lax.dot_general(mb, win_ref[...], cdims,
                          preferred_element_type=f32) + bi_ref[2:3, :]
    ghr = lax.dot_general(hb, whr_ref[...], cdims,
                          preferred_element_type=f32) + bh_ref[0:1, :]
    ghz = lax.dot_general(hb, whz_ref[...], cdims,
                          preferred_element_type=f32) + bh_ref[1:2, :]
    ghn = lax.dot_general(hb, whn_ref[...], cdims,
                          preferred_element_type=f32) + bh_ref[2:3, :]
    r = jax.nn.sigmoid(gir + ghr)
    z = jax.nn.sigmoid(giz + ghz)
    n = jnp.tanh(gin + r * ghn)
    out_ref[...] = (1.0 - z) * n + z * h


def _tc_call(h, rm2, W1, b1, W2, b2, W_ih, b_ih, W_hh, b_hh):
    bf16 = jnp.bfloat16
    W_ih = W_ih.astype(bf16)
    W_hh = W_hh.astype(bf16)
    wir, wiz, win = W_ih[:_D], W_ih[_D:2 * _D], W_ih[2 * _D:]
    whr, whz, whn = W_hh[:_D], W_hh[_D:2 * _D], W_hh[2 * _D:]
    bi = b_ih.reshape(3, _D)
    bh = b_hh.reshape(3, _D)
    full = lambda s: pl.BlockSpec(s, lambda i: (0, 0))
    return pl.pallas_call(
        _tc_body,
        grid=(_B // _BM,),
        in_specs=[
            pl.BlockSpec((_BM, 512), lambda i: (i, 0)),
            pl.BlockSpec((_BM, 128), lambda i: (i, 0)),
            full((_MD // 2, _MD)), full((1, _MD // 2)),
            full((_MD, _MD // 2)), full((1, _MD)),
            full((_D, _MD)), full((_D, _MD)), full((_D, _MD)), full((3, _D)),
            full((_D, _D)), full((_D, _D)), full((_D, _D)), full((3, _D)),
        ],
        out_specs=pl.BlockSpec((_BM, _D), lambda i: (i, 0)),
        out_shape=jax.ShapeDtypeStruct((_B, _D), jnp.float32),
    )(h, rm2, W1, b1.reshape(1, -1), W2, b2.reshape(1, -1),
      wir, wiz, win, bi, whr, whz, whn, bh)


def kernel(memory, node_idx, raw_messages, W1, b1, W2, b2,
           W_ih, b_ih, W_hh, b_hh):
    idx = node_idx.astype(jnp.int32)
    rm_p = jnp.pad(raw_messages, ((0, 0), (0, 128 - _MD)))
    mem512 = jnp.pad(memory, ((0, 0), (0, 512 - _D)))
    rm2 = _sc_winner()(rm_p, idx)
    h = _sc_hgather()(mem512, idx)
    return _tc_call(h, rm2, W1, b1, W2, b2, W_ih, b_ih, W_hh, b_hh)


# speedup vs baseline: 2.6758x; 2.6758x over previous
import functools

import jax
import jax.numpy as jnp
from jax import lax
from jax.experimental import pallas as pl
from jax.experimental.pallas import tpu as pltpu
from jax.experimental.pallas import tpu_sc as plsc

_N = 100000
_D = 500
_MD = 100
_B = 16384

_NC = 2
_NS = 16
_NW = _NC * _NS
_BPW = _B // _NW
_CRM = 32
_NRM = _BPW // _CRM
_SCAN = 1024
_NSB = _B // _SCAN
_CHB = 64
_NCHB = _BPW // _CHB

_SCMESH = dict(core_axis_name="c", subcore_axis_name="s",
               num_cores=_NC, num_subcores=_NS)


def _sc_winner_body(rm_hbm, idx_hbm, rm2_out, idx_v, w_v, tab, scan_v, rmbuf,
                    sem_s, sem_r, sem_rw):
    wid = lax.axis_index("s") * _NC + lax.axis_index("c")
    base = wid * _BPW
    pltpu.sync_copy(idx_hbm.at[pl.ds(base, _BPW)], idx_v)
    lanes = jnp.arange(16, dtype=jnp.int32)

    def _scan_compute(b, p):
        def _chunk(q, _):
            v = scan_v[p, pl.ds(q * 16, 16)]
            j = b * _SCAN + q * 16 + lanes
            plsc.store_scatter(tab, [v], j)

            def _wbody(c):
                got = plsc.load_gather(tab, [v])
                msk = got < j
                plsc.store_scatter(tab, [v], j, mask=msk)
                return plsc.all_reduce_population_count(msk)[0]

            lax.while_loop(lambda c: c > 0, _wbody, jnp.int32(1))
            return 0

        lax.fori_loop(0, _SCAN // 16, _chunk, 0)

    s_cps = {0: pltpu.async_copy(idx_hbm.at[pl.ds(0, _SCAN)],
                                 scan_v.at[0], sem_s)}
    for b in range(_NSB):
        s_cps.pop(b).wait()
        if b + 1 < _NSB:
            s_cps[b + 1] = pltpu.async_copy(
                idx_hbm.at[pl.ds((b + 1) * _SCAN, _SCAN)],
                scan_v.at[(b + 1) % 2], sem_s)
        _scan_compute(b, b % 2)

    def _wq(q, _):
        w_v[pl.ds(q * 16, 16)] = plsc.load_gather(
            tab, [idx_v[pl.ds(q * 16, 16)]])
        return 0

    lax.fori_loop(0, _BPW // 16, _wq, 0)

    def _issue_rm(c):
        return pltpu.async_copy(
            rm_hbm.at[w_v.at[pl.ds(c * _CRM, _CRM)]], rmbuf.at[c % 2], sem_r)

    r_cps = {0: _issue_rm(0)}
    rw_cps = {}
    for c in range(_NRM):
        if c + 1 < _NRM:
            r_cps[c + 1] = _issue_rm(c + 1)
        r_cps.pop(c).wait()
        rw_cps[c] = pltpu.async_copy(
            rmbuf.at[c % 2], rm2_out.at[pl.ds(base + c * _CRM, _CRM)], sem_rw)
        if c - 1 in rw_cps:
            rw_cps.pop(c - 1).wait()
    rw_cps.pop(_NRM - 1).wait()


@functools.cache
def _sc_winner():
    return pl.kernel(
        _sc_winner_body,
        out_type=jax.ShapeDtypeStruct((_B, 128), jnp.float32),
        mesh=plsc.VectorSubcoreMesh(**_SCMESH),
        compiler_params=pltpu.CompilerParams(needs_layout_passes=False),
        scratch_types=[
            pltpu.VMEM((_BPW,), jnp.int32),
            pltpu.VMEM((_BPW,), jnp.int32),
            pltpu.VMEM((_N,), jnp.int32),
            pltpu.VMEM((2, _SCAN), jnp.int32),
            pltpu.VMEM((2, _CRM, 128), jnp.float32),
            pltpu.SemaphoreType.DMA,
            pltpu.SemaphoreType.DMA,
            pltpu.SemaphoreType.DMA,
        ],
    )


def _sc_hgather_body(mem_hbm, idx_hbm, h_out, idx_v, hbuf, sem_g, sem_w):
    wid = lax.axis_index("s") * _NC + lax.axis_index("c")
    base = wid * _BPW
    pltpu.sync_copy(idx_hbm.at[pl.ds(base, _BPW)], idx_v)

    def _issue(c):
        ids = idx_v.at[pl.ds(c * _CHB, _CHB)]
        return [pltpu.async_copy(
                    mem_hbm.at[ids, pl.ds(k * 128, 128)],
                    hbuf.at[c % 2, :, pl.ds(k * 128, 128)], sem_g)
                for k in range(4)]

    g_cps = {0: _issue(0)}
    w_cps = {}
    for c in range(_NCHB):
        if c + 1 < _NCHB:
            g_cps[c + 1] = _issue(c + 1)
        for cp in g_cps.pop(c):
            cp.wait()
        w_cps[c] = pltpu.async_copy(
            hbuf.at[c % 2], h_out.at[pl.ds(base + c * _CHB, _CHB)], sem_w)
        if c - 1 in w_cps:
            w_cps.pop(c - 1).wait()
    w_cps.pop(_NCHB - 1).wait()


@functools.cache
def _sc_hgather():
    return pl.kernel(
        _sc_hgather_body,
        out_type=jax.ShapeDtypeStruct((_B, 512), jnp.float32),
        mesh=plsc.VectorSubcoreMesh(**_SCMESH),
        compiler_params=pltpu.CompilerParams(needs_layout_passes=False),
        scratch_types=[
            pltpu.VMEM((_BPW,), jnp.int32),
            pltpu.VMEM((2, _CHB, 512), jnp.float32),
            pltpu.SemaphoreType.DMA,
            pltpu.SemaphoreType.DMA,
        ],
    )


_TKN = 512


def _tr_body(mt_ref, out_ref):
    t = jnp.transpose(mt_ref[...], (1, 0))
    out_ref[...] = jnp.pad(t, ((0, 0), (0, 512 - _D)))


def _tr_call(mem_t):
    return pl.pallas_call(
        _tr_body,
        grid=(pl.cdiv(_N, _TKN),),
        in_specs=[pl.BlockSpec((_D, _TKN), lambda i: (0, i))],
        out_specs=pl.BlockSpec((_TKN, 512), lambda i: (i, 0)),
        out_shape=jax.ShapeDtypeStruct((_N, 512), jnp.float32),
    )(mem_t)


_BM = 256


def _tc_body(h_ref, rm_ref, w1_ref, b1_ref, w2_ref, b2_ref,
             wir_ref, wiz_ref, win_ref, bi_ref,
             whr_ref, whz_ref, whn_ref, bh_ref, out_ref):
    f32 = jnp.float32
    bf16 = jnp.bfloat16
    cdims = (((1,), (1,)), ((), ()))
    h = h_ref[:, :_D]
    hb = h.astype(bf16)
    m = jax.nn.relu(
        lax.dot_general(rm_ref[:, :_MD], w1_ref[...], cdims,
                        preferred_element_type=f32) + b1_ref[...])
    m = lax.dot_general(m, w2_ref[...], cdims,
                        preferred_element_type=f32) + b2_ref[...]
    mb = m.astype(bf16)
    gir = lax.dot_general(mb, wir_ref[...], cdims,
                          preferred_element_type=f32) + bi_ref[0:1, :]
    giz = lax.dot_general(mb, wiz_ref[...], cdims,
                          preferred_element_type=f32) + bi_ref[1:2, :]
    gin = lax.dot_general(mb, win_ref[...], cdims,
                          preferred_element_type=f32) + bi_ref[2:3, :]
    ghr = lax.dot_general(hb, whr_ref[...], cdims,
                          preferred_element_type=f32) + bh_ref[0:1, :]
    ghz = lax.dot_general(hb, whz_ref[...], cdims,
                          preferred_element_type=f32) + bh_ref[1:2, :]
    ghn = lax.dot_general(hb, whn_ref[...], cdims,
                          preferred_element_type=f32) + bh_ref[2:3, :]
    r = jax.nn.sigmoid(gir + ghr)
    z = jax.nn.sigmoid(giz + ghz)
    n = jnp.tanh(gin + r * ghn)
    out_ref[...] = (1.0 - z) * n + z * h


def _tc_call(h, rm2, W1, b1, W2, b2, W_ih, b_ih, W_hh, b_hh):
    bf16 = jnp.bfloat16
    W_ih = W_ih.astype(bf16)
    W_hh = W_hh.astype(bf16)
    wir, wiz, win = W_ih[:_D], W_ih[_D:2 * _D], W_ih[2 * _D:]
    whr, whz, whn = W_hh[:_D], W_hh[_D:2 * _D], W_hh[2 * _D:]
    bi = b_ih.reshape(3, _D)
    bh = b_hh.reshape(3, _D)
    full = lambda s: pl.BlockSpec(s, lambda i: (0, 0))
    return pl.pallas_call(
        _tc_body,
        grid=(_B // _BM,),
        in_specs=[
            pl.BlockSpec((_BM, 512), lambda i: (i, 0)),
            pl.BlockSpec((_BM, 128), lambda i: (i, 0)),
            full((_MD // 2, _MD)), full((1, _MD // 2)),
            full((_MD, _MD // 2)), full((1, _MD)),
            full((_D, _MD)), full((_D, _MD)), full((_D, _MD)), full((3, _D)),
            full((_D, _D)), full((_D, _D)), full((_D, _D)), full((3, _D)),
        ],
        out_specs=pl.BlockSpec((_BM, _D), lambda i: (i, 0)),
        out_shape=jax.ShapeDtypeStruct((_B, _D), jnp.float32),
    )(h, rm2, W1, b1.reshape(1, -1), W2, b2.reshape(1, -1),
      wir, wiz, win, bi, whr, whz, whn, bh)


def kernel(memory, node_idx, raw_messages, W1, b1, W2, b2,
           W_ih, b_ih, W_hh, b_hh):
    idx = node_idx.astype(jnp.int32)
    rm_p = jnp.pad(raw_messages, ((0, 0), (0, 128 - _MD)))
    mem512 = _tr_call(memory.T)
    rm2 = _sc_winner()(rm_p, idx)
    h = _sc_hgather()(mem512, idx)
    return _tc_call(h, rm2, W1, b1, W2, b2, W_ih, b_ih, W_hh, b_hh)


# speedup vs baseline: 2.8656x; 1.0709x over previous
import functools

import jax
import jax.numpy as jnp
from jax import lax
from jax.experimental import pallas as pl
from jax.experimental.pallas import tpu as pltpu
from jax.experimental.pallas import tpu_sc as plsc

_N = 100000
_D = 500
_MD = 100
_B = 16384

_NC = 2
_NS = 16
_NW = _NC * _NS
_BPW = _B // _NW
_CRM = 32
_NRM = _BPW // _CRM
_SCAN = 1024
_NSB = _B // _SCAN
_CHB = 64
_NCHB = _BPW // _CHB

_SCMESH = dict(core_axis_name="c", subcore_axis_name="s",
               num_cores=_NC, num_subcores=_NS)


def _sc_winner_body(rm_hbm, idx_hbm, rm2_out, w_out,
                    idx_v, w_v, tab, scan_v, rmbuf,
                    sem_s, sem_r, sem_rw):
    wid = lax.axis_index("s") * _NC + lax.axis_index("c")
    base = wid * _BPW
    pltpu.sync_copy(idx_hbm.at[pl.ds(base, _BPW)], idx_v)
    lanes = jnp.arange(16, dtype=jnp.int32)

    def _scan_compute(b, p):
        def _chunk(q, _):
            v = scan_v[p, pl.ds(q * 16, 16)]
            j = b * _SCAN + q * 16 + lanes
            plsc.store_scatter(tab, [v], j)

            def _wbody(c):
                got = plsc.load_gather(tab, [v])
                msk = got < j
                plsc.store_scatter(tab, [v], j, mask=msk)
                return plsc.all_reduce_population_count(msk)[0]

            lax.while_loop(lambda c: c > 0, _wbody, jnp.int32(1))
            return 0

        lax.fori_loop(0, _SCAN // 16, _chunk, 0)

    s_cps = {0: pltpu.async_copy(idx_hbm.at[pl.ds(0, _SCAN)],
                                 scan_v.at[0], sem_s)}
    for b in range(_NSB):
        s_cps.pop(b).wait()
        if b + 1 < _NSB:
            s_cps[b + 1] = pltpu.async_copy(
                idx_hbm.at[pl.ds((b + 1) * _SCAN, _SCAN)],
                scan_v.at[(b + 1) % 2], sem_s)
        _scan_compute(b, b % 2)

    def _wq(q, _):
        w_v[pl.ds(q * 16, 16)] = plsc.load_gather(
            tab, [idx_v[pl.ds(q * 16, 16)]])
        return 0

    lax.fori_loop(0, _BPW // 16, _wq, 0)
    pltpu.sync_copy(w_v, w_out.at[pl.ds(base, _BPW)])

    def _issue_rm(c):
        return pltpu.async_copy(
            rm_hbm.at[w_v.at[pl.ds(c * _CRM, _CRM)]], rmbuf.at[c % 2], sem_r)

    r_cps = {0: _issue_rm(0)}
    rw_cps = {}
    for c in range(_NRM):
        if c + 1 < _NRM:
            if c - 1 in rw_cps:
                rw_cps.pop(c - 1).wait()
            r_cps[c + 1] = _issue_rm(c + 1)
        r_cps.pop(c).wait()
        rw_cps[c] = pltpu.async_copy(
            rmbuf.at[c % 2], rm2_out.at[pl.ds(base + c * _CRM, _CRM)], sem_rw)
    for cp in rw_cps.values():
        cp.wait()


@functools.cache
def _sc_winner():
    return pl.kernel(
        _sc_winner_body,
        out_type=[jax.ShapeDtypeStruct((_B, 128), jnp.float32),
                  jax.ShapeDtypeStruct((_B,), jnp.int32)],
        mesh=plsc.VectorSubcoreMesh(**_SCMESH),
        compiler_params=pltpu.CompilerParams(needs_layout_passes=False),
        scratch_types=[
            pltpu.VMEM((_BPW,), jnp.int32),
            pltpu.VMEM((_BPW,), jnp.int32),
            pltpu.VMEM((_N,), jnp.int32),
            pltpu.VMEM((2, _SCAN), jnp.int32),
            pltpu.VMEM((2, _CRM, 128), jnp.float32),
            pltpu.SemaphoreType.DMA,
            pltpu.SemaphoreType.DMA,
            pltpu.SemaphoreType.DMA,
        ],
    )


def _sc_hgather_body(mem_hbm, idx_hbm, wdep_hbm, h_out, idx_v, hbuf,
                     sem_g, sem_w):
    del wdep_hbm
    wid = lax.axis_index("s") * _NC + lax.axis_index("c")
    base = wid * _BPW
    pltpu.sync_copy(idx_hbm.at[pl.ds(base, _BPW)], idx_v)

    def _issue(c):
        ids = idx_v.at[pl.ds(c * _CHB, _CHB)]
        return [pltpu.async_copy(
                    mem_hbm.at[ids, pl.ds(k * 128, 128)],
                    hbuf.at[c % 2, :, pl.ds(k * 128, 128)], sem_g)
                for k in range(4)]

    g_cps = {0: _issue(0)}
    w_cps = {}
    for c in range(_NCHB):
        if c + 1 < _NCHB:
            if c - 1 in w_cps:
                w_cps.pop(c - 1).wait()
            g_cps[c + 1] = _issue(c + 1)
        for cp in g_cps.pop(c):
            cp.wait()
        w_cps[c] = pltpu.async_copy(
            hbuf.at[c % 2], h_out.at[pl.ds(base + c * _CHB, _CHB)], sem_w)
    for cp in w_cps.values():
        cp.wait()


@functools.cache
def _sc_hgather():
    return pl.kernel(
        _sc_hgather_body,
        out_type=jax.ShapeDtypeStruct((_B, 512), jnp.float32),
        mesh=plsc.VectorSubcoreMesh(**_SCMESH),
        compiler_params=pltpu.CompilerParams(needs_layout_passes=False),
        scratch_types=[
            pltpu.VMEM((_BPW,), jnp.int32),
            pltpu.VMEM((2, _CHB, 512), jnp.float32),
            pltpu.SemaphoreType.DMA,
            pltpu.SemaphoreType.DMA,
        ],
    )


_TKN = 512


def _tr_body(mt_ref, out_ref):
    t = jnp.transpose(mt_ref[...], (1, 0))
    out_ref[...] = jnp.pad(t, ((0, 0), (0, 512 - _D)))


def _tr_call(mem_t):
    return pl.pallas_call(
        _tr_body,
        grid=(pl.cdiv(_N, _TKN),),
        in_specs=[pl.BlockSpec((_D, _TKN), lambda i: (0, i))],
        out_specs=pl.BlockSpec((_TKN, 512), lambda i: (i, 0)),
        out_shape=jax.ShapeDtypeStruct((_N, 512), jnp.float32),
    )(mem_t)


_BM = 256


def _tc_body(h_ref, rm_ref, w1_ref, b1_ref, w2_ref, b2_ref,
             wir_ref, wiz_ref, win_ref, bi_ref,
             whr_ref, whz_ref, whn_ref, bh_ref, out_ref):
    f32 = jnp.float32
    bf16 = jnp.bfloat16
    cdims = (((1,), (1,)), ((), ()))
    h = h_ref[:, :_D]
    hb = h.astype(bf16)
    m = jax.nn.relu(
        lax.dot_general(rm_ref[:, :_MD], w1_ref[...], cdims,
                        preferred_element_type=f32) + b1_ref[...])
    m = lax.dot_general(m, w2_ref[...], cdims,
                        preferred_element_type=f32) + b2_ref[...]
    mb = m.astype(bf16)
    gir = lax.dot_general(mb, wir_ref[...], cdims,
                          preferred_element_type=f32) + bi_ref[0:1, :]
    giz = lax.dot_general(mb, wiz_ref[...], cdims,
                          preferred_element_type=f32) + bi_ref[1:2, :]
    gin = lax.dot_general(mb, win_ref[...], cdims,
                          preferred_element_type=f32) + bi_ref[2:3, :]
    ghr = lax.dot_general(hb, whr_ref[...], cdims,
                          preferred_element_type=f32) + bh_ref[0:1, :]
    ghz = lax.dot_general(hb, whz_ref[...], cdims,
                          preferred_element_type=f32) + bh_ref[1:2, :]
    ghn = lax.dot_general(hb, whn_ref[...], cdims,
                          preferred_element_type=f32) + bh_ref[2:3, :]
    r = jax.nn.sigmoid(gir + ghr)
    z = jax.nn.sigmoid(giz + ghz)
    n = jnp.tanh(gin + r * ghn)
    out_ref[...] = (1.0 - z) * n + z * h


def _tc_call(h, rm2, W1, b1, W2, b2, W_ih, b_ih, W_hh, b_hh):
    bf16 = jnp.bfloat16
    W_ih = W_ih.astype(bf16)
    W_hh = W_hh.astype(bf16)
    wir, wiz, win = W_ih[:_D], W_ih[_D:2 * _D], W_ih[2 * _D:]
    whr, whz, whn = W_hh[:_D], W_hh[_D:2 * _D], W_hh[2 * _D:]
    bi = b_ih.reshape(3, _D)
    bh = b_hh.reshape(3, _D)
    full = lambda s: pl.BlockSpec(s, lambda i: (0, 0))
    return pl.pallas_call(
        _tc_body,
        grid=(_B // _BM,),
        in_specs=[
            pl.BlockSpec((_BM, 512), lambda i: (i, 0)),
            pl.BlockSpec((_BM, 128), lambda i: (i, 0)),
            full((_MD // 2, _MD)), full((1, _MD // 2)),
            full((_MD, _MD // 2)), full((1, _MD)),
            full((_D, _MD)), full((_D, _MD)), full((_D, _MD)), full((3, _D)),
            full((_D, _D)), full((_D, _D)), full((_D, _D)), full((3, _D)),
        ],
        out_specs=pl.BlockSpec((_BM, _D), lambda i: (i, 0)),
        out_shape=jax.ShapeDtypeStruct((_B, _D), jnp.float32),
    )(h, rm2, W1, b1.reshape(1, -1), W2, b2.reshape(1, -1),
      wir, wiz, win, bi, whr, whz, whn, bh)


def kernel(memory, node_idx, raw_messages, W1, b1, W2, b2,
           W_ih, b_ih, W_hh, b_hh):
    idx = node_idx.astype(jnp.int32)
    rm_p = jnp.pad(raw_messages, ((0, 0), (0, 128 - _MD)))
    mem512 = _tr_call(memory.T)
    rm2, w = _sc_winner()(rm_p, idx)
    h = _sc_hgather()(mem512, idx, w)
    return _tc_call(h, rm2, W1, b1, W2, b2, W_ih, b_ih, W_hh, b_hh)


# speedup vs baseline: 3.5243x; 1.2299x over previous
import functools

import jax
import jax.numpy as jnp
from jax import lax
from jax.experimental import pallas as pl
from jax.experimental.pallas import tpu as pltpu
from jax.experimental.pallas import tpu_sc as plsc

_N = 100000
_D = 500
_MD = 100
_B = 16384

_NC = 2
_NS = 16
_NW = _NC * _NS
_BPW = _B // _NW
_CRM = 32
_NRM = _BPW // _CRM
_SCAN = 1024
_NSB = _B // _SCAN
_CHB = 64
_NCHB = _BPW // _CHB

_SCMESH = dict(core_axis_name="c", subcore_axis_name="s",
               num_cores=_NC, num_subcores=_NS)


def _sc_winner_body(rm_hbm, idx_hbm, rm2_out, w_out,
                    idx_v, w_v, tab, scan_v, rmbuf,
                    sem_s, sem_r, sem_rw):
    wid = lax.axis_index("s") * _NC + lax.axis_index("c")
    base = wid * _BPW
    pltpu.sync_copy(idx_hbm.at[pl.ds(base, _BPW)], idx_v)
    lanes = jnp.arange(16, dtype=jnp.int32)

    def _scan_compute(b, p):
        def _chunk(q, _):
            v = scan_v[p, pl.ds(q * 16, 16)]
            j = b * _SCAN + q * 16 + lanes
            plsc.store_scatter(tab, [v], j)

            def _wbody(c):
                got = plsc.load_gather(tab, [v])
                msk = got < j
                plsc.store_scatter(tab, [v], j, mask=msk)
                return plsc.all_reduce_population_count(msk)[0]

            lax.while_loop(lambda c: c > 0, _wbody, jnp.int32(1))
            return 0

        lax.fori_loop(0, _SCAN // 16, _chunk, 0)

    s_cps = {0: pltpu.async_copy(idx_hbm.at[pl.ds(0, _SCAN)],
                                 scan_v.at[0], sem_s)}
    for b in range(_NSB):
        s_cps.pop(b).wait()
        if b + 1 < _NSB:
            s_cps[b + 1] = pltpu.async_copy(
                idx_hbm.at[pl.ds((b + 1) * _SCAN, _SCAN)],
                scan_v.at[(b + 1) % 2], sem_s)
        _scan_compute(b, b % 2)

    def _wq(q, _):
        w_v[pl.ds(q * 16, 16)] = plsc.load_gather(
            tab, [idx_v[pl.ds(q * 16, 16)]])
        return 0

    lax.fori_loop(0, _BPW // 16, _wq, 0)
    pltpu.sync_copy(w_v, w_out.at[pl.ds(base, _BPW)])

    def _issue_rm(c):
        return pltpu.async_copy(
            rm_hbm.at[w_v.at[pl.ds(c * _CRM, _CRM)]], rmbuf.at[c % 2], sem_r)

    r_cps = {0: _issue_rm(0)}
    rw_cps = {}
    for c in range(_NRM):
        if c + 1 < _NRM:
            if c - 1 in rw_cps:
                rw_cps.pop(c - 1).wait()
            r_cps[c + 1] = _issue_rm(c + 1)
        r_cps.pop(c).wait()
        rw_cps[c] = pltpu.async_copy(
            rmbuf.at[c % 2], rm2_out.at[pl.ds(base + c * _CRM, _CRM)], sem_rw)
    for cp in rw_cps.values():
        cp.wait()


@functools.cache
def _sc_winner():
    return pl.kernel(
        _sc_winner_body,
        out_type=[jax.ShapeDtypeStruct((_B, 128), jnp.float32),
                  jax.ShapeDtypeStruct((_B,), jnp.int32)],
        mesh=plsc.VectorSubcoreMesh(**_SCMESH),
        compiler_params=pltpu.CompilerParams(needs_layout_passes=False),
        scratch_types=[
            pltpu.VMEM((_BPW,), jnp.int32),
            pltpu.VMEM((_BPW,), jnp.int32),
            pltpu.VMEM((_N,), jnp.int32),
            pltpu.VMEM((2, _SCAN), jnp.int32),
            pltpu.VMEM((2, _CRM, 128), jnp.float32),
            pltpu.SemaphoreType.DMA,
            pltpu.SemaphoreType.DMA,
            pltpu.SemaphoreType.DMA,
        ],
    )


def _sc_hgather_body(mem_hbm, idx_hbm, wdep_hbm, h_out, idx_v, hbuf,
                     sem_g, sem_w):
    del wdep_hbm
    wid = lax.axis_index("s") * _NC + lax.axis_index("c")
    base = wid * _BPW
    pltpu.sync_copy(idx_hbm.at[pl.ds(base, _BPW)], idx_v)

    def _issue(c):
        ids = idx_v.at[pl.ds(c * _CHB, _CHB)]
        return [pltpu.async_copy(
                    mem_hbm.at[ids, pl.ds(k * 128, 128)],
                    hbuf.at[c % 2, :, pl.ds(k * 128, 128)], sem_g)
                for k in range(4)]

    g_cps = {0: _issue(0)}
    w_cps = {}
    for c in range(_NCHB):
        if c + 1 < _NCHB:
            if c - 1 in w_cps:
                w_cps.pop(c - 1).wait()
            g_cps[c + 1] = _issue(c + 1)
        for cp in g_cps.pop(c):
            cp.wait()
        w_cps[c] = pltpu.async_copy(
            hbuf.at[c % 2], h_out.at[pl.ds(base + c * _CHB, _CHB)], sem_w)
    for cp in w_cps.values():
        cp.wait()


@functools.cache
def _sc_hgather():
    return pl.kernel(
        _sc_hgather_body,
        out_type=jax.ShapeDtypeStruct((_B, 512), jnp.float32),
        mesh=plsc.VectorSubcoreMesh(**_SCMESH),
        compiler_params=pltpu.CompilerParams(needs_layout_passes=False),
        scratch_types=[
            pltpu.VMEM((_BPW,), jnp.int32),
            pltpu.VMEM((2, _CHB, 512), jnp.float32),
            pltpu.SemaphoreType.DMA,
            pltpu.SemaphoreType.DMA,
        ],
    )


_TKN = 2048


def _tr_body(mt_ref, out_ref):
    t = jnp.transpose(mt_ref[...], (1, 0))
    out_ref[...] = jnp.pad(t, ((0, 0), (0, 512 - _D)))


def _tr_call(mem_t):
    return pl.pallas_call(
        _tr_body,
        grid=(pl.cdiv(_N, _TKN),),
        in_specs=[pl.BlockSpec((_D, _TKN), lambda i: (0, i))],
        out_specs=pl.BlockSpec((_TKN, 512), lambda i: (i, 0)),
        out_shape=jax.ShapeDtypeStruct((_N, 512), jnp.float32),
    )(mem_t)


_BM = 256


def _tc_body(h_ref, rm_ref, w1_ref, b1_ref, w2_ref, b2_ref,
             wir_ref, wiz_ref, win_ref, bi_ref,
             whr_ref, whz_ref, whn_ref, bh_ref, out_ref):
    f32 = jnp.float32
    bf16 = jnp.bfloat16
    cdims = (((1,), (1,)), ((), ()))
    h = h_ref[:, :_D]
    hb = h.astype(bf16)
    m = jax.nn.relu(
        lax.dot_general(rm_ref[:, :_MD], w1_ref[...], cdims,
                        preferred_element_type=f32) + b1_ref[...])
    m = lax.dot_general(m, w2_ref[...], cdims,
                        preferred_element_type=f32) + b2_ref[...]
    mb = m.astype(bf16)
    gir = lax.dot_general(mb, wir_ref[...], cdims,
                          preferred_element_type=f32) + bi_ref[0:1, :]
    giz = lax.dot_general(mb, wiz_ref[...], cdims,
                          preferred_element_type=f32) + bi_ref[1:2, :]
    gin = lax.dot_general(mb, win_ref[...], cdims,
                          preferred_element_type=f32) + bi_ref[2:3, :]
    ghr = lax.dot_general(hb, whr_ref[...], cdims,
                          preferred_element_type=f32) + bh_ref[0:1, :]
    ghz = lax.dot_general(hb, whz_ref[...], cdims,
                          preferred_element_type=f32) + bh_ref[1:2, :]
    ghn = lax.dot_general(hb, whn_ref[...], cdims,
                          preferred_element_type=f32) + bh_ref[2:3, :]
    r = jax.nn.sigmoid(gir + ghr)
    z = jax.nn.sigmoid(giz + ghz)
    n = jnp.tanh(gin + r * ghn)
    out_ref[...] = (1.0 - z) * n + z * h


def _tc_call(h, rm2, W1, b1, W2, b2, W_ih, b_ih, W_hh, b_hh):
    bf16 = jnp.bfloat16
    W_ih = W_ih.astype(bf16)
    W_hh = W_hh.astype(bf16)
    wir, wiz, win = W_ih[:_D], W_ih[_D:2 * _D], W_ih[2 * _D:]
    whr, whz, whn = W_hh[:_D], W_hh[_D:2 * _D], W_hh[2 * _D:]
    bi = b_ih.reshape(3, _D)
    bh = b_hh.reshape(3, _D)
    full = lambda s: pl.BlockSpec(s, lambda i: (0, 0))
    return pl.pallas_call(
        _tc_body,
        grid=(_B // _BM,),
        in_specs=[
            pl.BlockSpec((_BM, 512), lambda i: (i, 0)),
            pl.BlockSpec((_BM, 128), lambda i: (i, 0)),
            full((_MD // 2, _MD)), full((1, _MD // 2)),
            full((_MD, _MD // 2)), full((1, _MD)),
            full((_D, _MD)), full((_D, _MD)), full((_D, _MD)), full((3, _D)),
            full((_D, _D)), full((_D, _D)), full((_D, _D)), full((3, _D)),
        ],
        out_specs=pl.BlockSpec((_BM, _D), lambda i: (i, 0)),
        out_shape=jax.ShapeDtypeStruct((_B, _D), jnp.float32),
    )(h, rm2, W1, b1.reshape(1, -1), W2, b2.reshape(1, -1),
      wir, wiz, win, bi, whr, whz, whn, bh)


def kernel(memory, node_idx, raw_messages, W1, b1, W2, b2,
           W_ih, b_ih, W_hh, b_hh):
    idx = node_idx.astype(jnp.int32)
    rm_p = jnp.pad(raw_messages, ((0, 0), (0, 128 - _MD)))
    mem512 = _tr_call(memory.T)
    rm2, w = _sc_winner()(rm_p, idx)
    h = _sc_hgather()(mem512, idx, w)
    return _tc_call(h, rm2, W1, b1, W2, b2, W_ih, b_ih, W_hh, b_hh)


# speedup vs baseline: 3.6179x; 1.0266x over previous
import functools

import jax
import jax.numpy as jnp
from jax import lax
from jax.experimental import pallas as pl
from jax.experimental.pallas import tpu as pltpu
from jax.experimental.pallas import tpu_sc as plsc

_N = 100000
_D = 500
_MD = 100
_B = 16384

_NC = 2
_NS = 16
_NW = _NC * _NS
_BPW = _B // _NW
_CRM = 32
_NRM = _BPW // _CRM
_SCAN = 1024
_NSB = _B // _SCAN
_CHB = 64
_NCHB = _BPW // _CHB

_SCMESH = dict(core_axis_name="c", subcore_axis_name="s",
               num_cores=_NC, num_subcores=_NS)


def _sc_winner_body(rm_hbm, idx_hbm, rm2_out, w_out,
                    idx_v, w_v, tab, scan_v, rmbuf,
                    sem_s, sem_r, sem_rw):
    wid = lax.axis_index("s") * _NC + lax.axis_index("c")
    base = wid * _BPW
    pltpu.sync_copy(idx_hbm.at[pl.ds(base, _BPW)], idx_v)
    lanes = jnp.arange(16, dtype=jnp.int32)

    def _scan_compute(b, p):
        def _chunk(q, _):
            v = scan_v[p, pl.ds(q * 16, 16)]
            j = b * _SCAN + q * 16 + lanes
            plsc.store_scatter(tab, [v], j)

            def _wbody(c):
                got = plsc.load_gather(tab, [v])
                msk = got < j
                plsc.store_scatter(tab, [v], j, mask=msk)
                return plsc.all_reduce_population_count(msk)[0]

            lax.while_loop(lambda c: c > 0, _wbody, jnp.int32(1))
            return 0

        lax.fori_loop(0, _SCAN // 16, _chunk, 0)

    s_cps = {0: pltpu.async_copy(idx_hbm.at[pl.ds(0, _SCAN)],
                                 scan_v.at[0], sem_s)}
    for b in range(_NSB):
        s_cps.pop(b).wait()
        if b + 1 < _NSB:
            s_cps[b + 1] = pltpu.async_copy(
                idx_hbm.at[pl.ds((b + 1) * _SCAN, _SCAN)],
                scan_v.at[(b + 1) % 2], sem_s)
        _scan_compute(b, b % 2)

    def _wq(q, _):
        w_v[pl.ds(q * 16, 16)] = plsc.load_gather(
            tab, [idx_v[pl.ds(q * 16, 16)]])
        return 0

    lax.fori_loop(0, _BPW // 16, _wq, 0)
    pltpu.sync_copy(w_v, w_out.at[pl.ds(base, _BPW)])

    def _issue_rm(c):
        return pltpu.async_copy(
            rm_hbm.at[w_v.at[pl.ds(c * _CRM, _CRM)]], rmbuf.at[c % 2], sem_r)

    r_cps = {0: _issue_rm(0)}
    rw_cps = {}
    for c in range(_NRM):
        if c + 1 < _NRM:
            if c - 1 in rw_cps:
                rw_cps.pop(c - 1).wait()
            r_cps[c + 1] = _issue_rm(c + 1)
        r_cps.pop(c).wait()
        rw_cps[c] = pltpu.async_copy(
            rmbuf.at[c % 2], rm2_out.at[pl.ds(base + c * _CRM, _CRM)], sem_rw)
    for cp in rw_cps.values():
        cp.wait()


@functools.cache
def _sc_winner():
    return pl.kernel(
        _sc_winner_body,
        out_type=[jax.ShapeDtypeStruct((_B, 128), jnp.float32),
                  jax.ShapeDtypeStruct((_B,), jnp.int32)],
        mesh=plsc.VectorSubcoreMesh(**_SCMESH),
        compiler_params=pltpu.CompilerParams(needs_layout_passes=False),
        scratch_types=[
            pltpu.VMEM((_BPW,), jnp.int32),
            pltpu.VMEM((_BPW,), jnp.int32),
            pltpu.VMEM((_N,), jnp.int32),
            pltpu.VMEM((2, _SCAN), jnp.int32),
            pltpu.VMEM((2, _CRM, 128), jnp.float32),
            pltpu.SemaphoreType.DMA,
            pltpu.SemaphoreType.DMA,
            pltpu.SemaphoreType.DMA,
        ],
    )


def _sc_hgather_body(mem_hbm, idx_hbm, wdep_hbm, h_out, idx_v, hbuf,
                     sem_g, sem_w):
    del wdep_hbm
    wid = lax.axis_index("s") * _NC + lax.axis_index("c")
    base = wid * _BPW
    pltpu.sync_copy(idx_hbm.at[pl.ds(base, _BPW)], idx_v)

    def _issue(c):
        ids = idx_v.at[pl.ds(c * _CHB, _CHB)]
        return [pltpu.async_copy(
                    mem_hbm.at[ids, pl.ds(k * 128, 128)],
                    hbuf.at[c % 2, :, pl.ds(k * 128, 128)], sem_g)
                for k in range(4)]

    g_cps = {0: _issue(0)}
    w_cps = {}
    for c in range(_NCHB):
        if c + 1 < _NCHB:
            if c - 1 in w_cps:
                w_cps.pop(c - 1).wait()
            g_cps[c + 1] = _issue(c + 1)
        for cp in g_cps.pop(c):
            cp.wait()
        w_cps[c] = pltpu.async_copy(
            hbuf.at[c % 2], h_out.at[pl.ds(base + c * _CHB, _CHB)], sem_w)
    for cp in w_cps.values():
        cp.wait()


@functools.cache
def _sc_hgather():
    return pl.kernel(
        _sc_hgather_body,
        out_type=jax.ShapeDtypeStruct((_B, 512), jnp.float32),
        mesh=plsc.VectorSubcoreMesh(**_SCMESH),
        compiler_params=pltpu.CompilerParams(needs_layout_passes=False),
        scratch_types=[
            pltpu.VMEM((_BPW,), jnp.int32),
            pltpu.VMEM((2, _CHB, 512), jnp.float32),
            pltpu.SemaphoreType.DMA,
            pltpu.SemaphoreType.DMA,
        ],
    )


_TKN = 4096


def _tr_body(mt_ref, out_ref):
    t = jnp.transpose(mt_ref[...], (1, 0))
    out_ref[...] = jnp.pad(t, ((0, 0), (0, 512 - _D)))


def _tr_call(mem_t):
    return pl.pallas_call(
        _tr_body,
        grid=(pl.cdiv(_N, _TKN),),
        in_specs=[pl.BlockSpec((_D, _TKN), lambda i: (0, i))],
        out_specs=pl.BlockSpec((_TKN, 512), lambda i: (i, 0)),
        out_shape=jax.ShapeDtypeStruct((_N, 512), jnp.float32),
    )(mem_t)


_BM = 256


def _tc_body(h_ref, rm_ref, w1_ref, b1_ref, w2_ref, b2_ref,
             wir_ref, wiz_ref, win_ref, bi_ref,
             whr_ref, whz_ref, whn_ref, bh_ref, out_ref):
    f32 = jnp.float32
    bf16 = jnp.bfloat16
    cdims = (((1,), (1,)), ((), ()))
    h = h_ref[:, :_D]
    hb = h.astype(bf16)
    m = jax.nn.relu(
        lax.dot_general(rm_ref[:, :_MD], w1_ref[...], cdims,
                        preferred_element_type=f32) + b1_ref[...])
    m = lax.dot_general(m, w2_ref[...], cdims,
                        preferred_element_type=f32) + b2_ref[...]
    mb = m.astype(bf16)
    gir = lax.dot_general(mb, wir_ref[...], cdims,
                          preferred_element_type=f32) + bi_ref[0:1, :]
    giz = lax.dot_general(mb, wiz_ref[...], cdims,
                          preferred_element_type=f32) + bi_ref[1:2, :]
    gin = lax.dot_general(mb, win_ref[...], cdims,
                          preferred_element_type=f32) + bi_ref[2:3, :]
    ghr = lax.dot_general(hb, whr_ref[...], cdims,
                          preferred_element_type=f32) + bh_ref[0:1, :]
    ghz = lax.dot_general(hb, whz_ref[...], cdims,
                          preferred_element_type=f32) + bh_ref[1:2, :]
    ghn = lax.dot_general(hb, whn_ref[...], cdims,
                          preferred_element_type=f32) + bh_ref[2:3, :]
    r = jax.nn.sigmoid(gir + ghr)
    z = jax.nn.sigmoid(giz + ghz)
    n = jnp.tanh(gin + r * ghn)
    out_ref[...] = (1.0 - z) * n + z * h


def _tc_call(h, rm2, W1, b1, W2, b2, W_ih, b_ih, W_hh, b_hh):
    bf16 = jnp.bfloat16
    W_ih = W_ih.astype(bf16)
    W_hh = W_hh.astype(bf16)
    wir, wiz, win = W_ih[:_D], W_ih[_D:2 * _D], W_ih[2 * _D:]
    whr, whz, whn = W_hh[:_D], W_hh[_D:2 * _D], W_hh[2 * _D:]
    bi = b_ih.reshape(3, _D)
    bh = b_hh.reshape(3, _D)
    full = lambda s: pl.BlockSpec(s, lambda i: (0, 0))
    return pl.pallas_call(
        _tc_body,
        grid=(_B // _BM,),
        in_specs=[
            pl.BlockSpec((_BM, 512), lambda i: (i, 0)),
            pl.BlockSpec((_BM, 128), lambda i: (i, 0)),
            full((_MD // 2, _MD)), full((1, _MD // 2)),
            full((_MD, _MD // 2)), full((1, _MD)),
            full((_D, _MD)), full((_D, _MD)), full((_D, _MD)), full((3, _D)),
            full((_D, _D)), full((_D, _D)), full((_D, _D)), full((3, _D)),
        ],
        out_specs=pl.BlockSpec((_BM, _D), lambda i: (i, 0)),
        out_shape=jax.ShapeDtypeStruct((_B, _D), jnp.float32),
    )(h, rm2, W1, b1.reshape(1, -1), W2, b2.reshape(1, -1),
      wir, wiz, win, bi, whr, whz, whn, bh)


def kernel(memory, node_idx, raw_messages, W1, b1, W2, b2,
           W_ih, b_ih, W_hh, b_hh):
    idx = node_idx.astype(jnp.int32)
    rm_p = jnp.pad(raw_messages, ((0, 0), (0, 128 - _MD)))
    mem512 = _tr_call(memory.T)
    rm2, w = _sc_winner()(rm_p, idx)
    h = _sc_hgather()(mem512, idx, w)
    return _tc_call(h, rm2, W1, b1, W2, b2, W_ih, b_ih, W_hh, b_hh)
